# baseline (speedup 1.0000x reference)
import functools

import jax
import jax.numpy as jnp
from jax import lax
from jax.experimental import pallas as pl
from jax.experimental.pallas import tpu as pltpu
from jax.experimental.pallas import tpu_sc as plsc

_N = 65536
_D = 768
_NW = 32
_BPW = _N // _NW
_W = 64
_NCHUNK = _BPW // _W

_mesh = plsc.VectorSubcoreMesh(core_axis_name="core", subcore_axis_name="subcore")


@jax.jit
def _sc_gather(table, idx):
    @functools.partial(
        pl.kernel,
        out_type=jax.ShapeDtypeStruct((_N, _D), table.dtype),
        mesh=_mesh,
        scratch_types=[
            pltpu.VMEM((_BPW,), jnp.int32),
            pltpu.VMEM((_W, _D), jnp.float32),
            pltpu.SemaphoreType.DMA,
        ],
    )
    def k(table_hbm, idx_hbm, out_hbm, idx_v, rows_v, sem):
        wid = lax.axis_index("subcore") * 2 + lax.axis_index("core")
        base = wid * _BPW
        pltpu.sync_copy(idx_hbm.at[pl.ds(base, _BPW)], idx_v)

        @pl.loop(0, _NCHUNK)
        def _(c):
            off = c * _W
            pltpu.async_copy(
                table_hbm.at[idx_v.at[pl.ds(off, _W)]], rows_v, sem
            ).wait()
            pltpu.sync_copy(rows_v, out_hbm.at[pl.ds(base + off, _W)])

    return k(table, idx)


def kernel(mark_indices, marks_weight):
    return _sc_gather(marks_weight, mark_indices.astype(jnp.int32))


# speedup vs baseline: 1.0063x; 1.0063x over previous
import functools

import jax
import jax.numpy as jnp
from jax import lax
from jax.experimental import pallas as pl
from jax.experimental.pallas import tpu as pltpu
from jax.experimental.pallas import tpu_sc as plsc

_NM = 16
_N = 65536
_D = 768
_NW = 32
_BPW = _N // _NW
_W = 64
_NBUF = 2
_NCHUNK = _BPW // _W
_NROUND = _NCHUNK // _NBUF

_mesh = plsc.VectorSubcoreMesh(core_axis_name="core", subcore_axis_name="subcore")


@jax.jit
def _sc_gather(table, idx):
    @functools.partial(
        pl.kernel,
        out_type=jax.ShapeDtypeStruct((_N, _D), table.dtype),
        mesh=_mesh,
        scratch_types=[
            pltpu.VMEM((_BPW,), jnp.int32),
            pltpu.VMEM((_NBUF, _W, _D), jnp.float32),
            pltpu.SemaphoreType.DMA((_NBUF,)),
            pltpu.SemaphoreType.DMA((_NBUF,)),
        ],
    )
    def k(table_hbm, idx_hbm, out_hbm, idx_v, bufs, gsem, wsem):
        wid = lax.axis_index("subcore") * 2 + lax.axis_index("core")
        base = wid * _BPW
        pltpu.sync_copy(idx_hbm.at[pl.ds(base, _BPW)], idx_v)

        def gather(c_off, b):
            pltpu.async_copy(
                table_hbm.at[idx_v.at[pl.ds(c_off, _W)]], bufs.at[b], gsem.at[b]
            )

        def wait_gather(b):
            pltpu.make_async_copy(
                table_hbm.at[idx_v.at[pl.ds(0, _W)]], bufs.at[b], gsem.at[b]
            ).wait()

        def write(c_off, b):
            pltpu.async_copy(
                bufs.at[b], out_hbm.at[pl.ds(base + c_off, _W)], wsem.at[b]
            )

        def wait_write(b):
            pltpu.make_async_copy(
                bufs.at[b], out_hbm.at[pl.ds(base, _W)], wsem.at[b]
            ).wait()

        for b in range(_NBUF):
            gather(b * _W, b)

        def do_round(g, regather):
            for b in range(_NBUF):
                off = (g * _NBUF + b) * _W
                wait_gather(b)
                write(off, b)
                if regather:
                    wait_write(b)
                    gather(off + _NBUF * _W, b)

        @pl.loop(0, _NROUND - 1)
        def _(g):
            do_round(g, True)

        do_round(_NROUND - 1, False)
        for b in range(_NBUF):
            wait_write(b)

    return k(table, idx)


def kernel(mark_indices, marks_weight):
    return _sc_gather(marks_weight, mark_indices.astype(jnp.int32))


# speedup vs baseline: 7.0009x; 6.9571x over previous
import functools

import jax
import jax.numpy as jnp
from jax import lax
from jax.experimental import pallas as pl
from jax.experimental.pallas import tpu as pltpu
from jax.experimental.pallas import tpu_sc as plsc

_NM = 16
_N = 65536
_D = 768
_NW = 32
_BPW = _N // _NW
_SW = 512

_mesh = plsc.VectorSubcoreMesh(core_axis_name="core", subcore_axis_name="subcore")


@jax.jit
def _sc_gather(table, idx):
    @functools.partial(
        pl.kernel,
        out_type=jax.ShapeDtypeStruct((_N, _D), table.dtype),
        mesh=_mesh,
        scratch_types=[
            pltpu.VMEM((_BPW,), jnp.int32),
            pltpu.VMEM((_NM, _D), jnp.float32),
            pltpu.SemaphoreType.DMA,
        ],
    )
    def k(table_hbm, idx_hbm, out_hbm, idx_v, table_v, sem):
        wid = lax.axis_index("subcore") * 2 + lax.axis_index("core")
        base = wid * _BPW
        pltpu.sync_copy(table_hbm, table_v)
        pltpu.sync_copy(idx_hbm.at[pl.ds(base, _BPW)], idx_v)

        @pl.loop(0, _BPW, step=16)
        def _(r16):
            v = idx_v[pl.ds(r16, 16)]
            for j in range(16):
                s = v[j]
                pltpu.async_copy(
                    table_v.at[pl.ds(s, 1)],
                    out_hbm.at[pl.ds(base + r16 + j, 1)],
                    sem,
                )

        pltpu.make_async_copy(
            out_hbm.at[pl.ds(base, _BPW)], out_hbm.at[pl.ds(base, _BPW)], sem
        ).wait()

    return k(table, idx)


def kernel(mark_indices, marks_weight):
    return _sc_gather(marks_weight, mark_indices.astype(jnp.int32))
